# Initial kernel scaffold; baseline (speedup 1.0000x reference)
import jax
import jax.numpy as jnp
from jax.experimental import pallas as pl


def kernel(x, gate_up_proj, down_proj, topk_weights, topk_ids):
    raise NotImplementedError("write your pallas kernel here")


# speedup vs baseline: 2.1107x; 2.1107x over previous
import functools

import jax
import jax.numpy as jnp
from jax.experimental import pallas as pl
from jax.experimental.pallas import tpu as pltpu


BT = 512
F = 1024


def _mm_body(be_ref, brow_ref, bval_ref, xs_ref, gate_ref, up_ref, down_ref,
             w_ref, out_ref):
    g = pl.program_id(0)
    j = pl.program_id(1)

    @pl.when(bval_ref[g] == 1)
    def _():
        x_ = xs_ref[...]
        dn = (((1,), (1,)), ((), ()))
        gt = jax.lax.dot_general(x_, gate_ref[0], dn,
                                 preferred_element_type=jnp.float32)
        up = jax.lax.dot_general(x_, up_ref[0], dn,
                                 preferred_element_type=jnp.float32)
        h = gt * jax.nn.sigmoid(gt) * up
        y = jax.lax.dot_general(h, down_ref[0], dn,
                                preferred_element_type=jnp.float32)
        y = y * w_ref[...]

        @pl.when(j == 0)
        def _():
            out_ref[...] = y

        @pl.when(j != 0)
        def _():
            out_ref[...] += y


def _grouped_matmul(xs, gate_up, down, w_pad, block_expert, block_row,
                    block_valid, g_max, nf):
    n_rows, d_model = xs.shape
    e, ff2, _ = gate_up.shape
    ff = ff2 // 2

    grid_spec = pltpu.PrefetchScalarGridSpec(
        num_scalar_prefetch=3,
        grid=(g_max, nf),
        in_specs=[
            pl.BlockSpec((BT, d_model),
                         lambda g, j, be, br, bv: (br[g], 0)),
            pl.BlockSpec((1, F, d_model),
                         lambda g, j, be, br, bv: (be[g], j, 0)),
            pl.BlockSpec((1, F, d_model),
                         lambda g, j, be, br, bv: (be[g], (ff // F) + j, 0)),
            pl.BlockSpec((1, d_model, F),
                         lambda g, j, be, br, bv: (be[g], 0, j)),
            pl.BlockSpec((BT, 1),
                         lambda g, j, be, br, bv: (br[g], 0)),
        ],
        out_specs=pl.BlockSpec((BT, d_model),
                               lambda g, j, be, br, bv: (br[g], 0)),
    )
    return pl.pallas_call(
        _mm_body,
        grid_spec=grid_spec,
        out_shape=jax.ShapeDtypeStruct((n_rows, d_model), jnp.float32),
        compiler_params=pltpu.CompilerParams(
            dimension_semantics=("arbitrary", "arbitrary"),
        ),
    )(block_expert, block_row, block_valid, xs, gate_up, gate_up, down, w_pad)


def kernel(x, gate_up_proj, down_proj, topk_weights, topk_ids):
    t, d_model = x.shape
    e = gate_up_proj.shape[0]
    k = topk_ids.shape[1]
    n = t * k

    g_max = -(-n // BT) + e - 1
    g_max = -(-g_max // 4) * 4
    n_rows = g_max * BT
    nf = (gate_up_proj.shape[1] // 2) // F

    flat = topk_ids.reshape(-1).astype(jnp.int32)
    order = jnp.argsort(flat)
    flat_sorted = flat[order]
    tok_sorted = (order // k).astype(jnp.int32)

    counts = jnp.bincount(flat, length=e)
    blocks_e = -(-counts // BT)
    block_start = jnp.concatenate(
        [jnp.zeros((1,), jnp.int32), jnp.cumsum(blocks_e).astype(jnp.int32)])
    total_blocks = block_start[e]

    g_ids = jnp.arange(g_max, dtype=jnp.int32)
    be = jnp.searchsorted(block_start[1:], g_ids, side="right").astype(jnp.int32)
    be = jnp.minimum(be, e - 1)
    block_valid = (g_ids < total_blocks).astype(jnp.int32)
    last_valid = jnp.maximum(total_blocks - 1, 0)
    block_expert = jnp.where(block_valid == 1, be, be[last_valid])
    block_row = jnp.where(block_valid == 1, g_ids, last_valid)

    cnt_start = jnp.concatenate(
        [jnp.zeros((1,), jnp.int32), jnp.cumsum(counts).astype(jnp.int32)])
    pad_off = block_start[:e] * BT
    p = pad_off[flat_sorted] + (jnp.arange(n, dtype=jnp.int32)
                                - cnt_start[flat_sorted])

    row_tok = jnp.zeros((n_rows,), jnp.int32).at[p].set(tok_sorted)
    w_sorted = topk_weights.reshape(-1)[order]
    w_pad = jnp.zeros((n_rows,), jnp.float32).at[p].set(w_sorted)
    dest = jnp.zeros((n,), jnp.int32).at[order].set(p)
    dest = dest.reshape(t, k)

    xs = jnp.take(x, row_tok, axis=0)

    ys = _grouped_matmul(xs, gate_up_proj, down_proj,
                         w_pad.reshape(n_rows, 1),
                         block_expert, block_row, block_valid, g_max, nf)

    out = jnp.take(ys, dest[:, 0], axis=0)
    for kk in range(1, k):
        out = out + jnp.take(ys, dest[:, kk], axis=0)
    return out
